# baseline (speedup 1.0000x reference)
import functools

import jax
import jax.numpy as jnp
from jax import lax
from jax.experimental import pallas as pl
from jax.experimental.pallas import tpu as pltpu
from jax.experimental.pallas import tpu_sc as plsc

N = 10000
H = 128
L = 4
G = 64
C = 10

NC = 2
NS = 16
CHUNK = 128
NP = 10112
NCH = NP // CHUNK
ZPT = 5
RB = 1000
EPS = 1e-5


@functools.lru_cache(maxsize=None)
def _make_sc_agg(cpt: int):
    mesh = plsc.VectorSubcoreMesh(core_axis_name="c", subcore_axis_name="s")
    hc = cpt // 2
    scratch = [
        pltpu.VMEM((hc, CHUNK), jnp.int32),
        pltpu.VMEM((hc, CHUNK), jnp.int32),
        pltpu.VMEM((CHUNK, H), jnp.float32),
        pltpu.VMEM((CHUNK, H), jnp.float32),
        pltpu.VMEM_SHARED((NP, H), jnp.float32),
        pltpu.SemaphoreType.DMA,
        pltpu.SemaphoreType.DMA,
    ]

    def body(h_hbm, src_hbm, dst_hbm, agg_hbm, src_v, dst_v, rows0, rows1,
             acc, sem0, sem1):
        c = lax.axis_index("c")
        s = lax.axis_index("s")
        wid = c * NS + s

        zv = jnp.zeros((16,), jnp.float32)

        def zero_row(i, carry):
            for q in range(H // 16):
                rows0[i, pl.ds(q * 16, 16)] = zv
            return carry

        lax.fori_loop(0, CHUNK, zero_row, 0)
        for k in range(ZPT):
            ci = s * ZPT + k

            @pl.when(ci < NCH)
            def _():
                pltpu.sync_copy(rows0, acc.at[pl.ds(ci * CHUNK, CHUNK)])

        plsc.subcore_barrier()

        def step(i, carry):
            a = 2 * i
            b = a + 1
            pltpu.async_copy(h_hbm.at[src_v.at[b]], rows1, sem1)
            pltpu.make_async_copy(h_hbm.at[src_v.at[a]], rows0, sem0).wait()
            pltpu.sync_copy(rows0, acc.at[dst_v.at[a]], add=True)

            @pl.when(b + 1 < hc)
            def _():
                pltpu.async_copy(h_hbm.at[src_v.at[b + 1]], rows0, sem0)

            pltpu.make_async_copy(h_hbm.at[src_v.at[b]], rows1, sem1).wait()
            pltpu.sync_copy(rows1, acc.at[dst_v.at[b]], add=True)
            return carry

        for p in range(2):
            pltpu.sync_copy(src_hbm.at[pl.ds(wid * cpt + p * hc, hc)], src_v)
            pltpu.sync_copy(dst_hbm.at[pl.ds(wid * cpt + p * hc, hc)], dst_v)
            pltpu.async_copy(h_hbm.at[src_v.at[0]], rows0, sem0)
            lax.fori_loop(0, hc // 2, step, 0)

        plsc.subcore_barrier()

        for k in range(ZPT):
            ci = s * ZPT + k

            @pl.when(ci < NCH)
            def _():
                pltpu.sync_copy(
                    acc.at[pl.ds(ci * CHUNK, CHUNK)],
                    agg_hbm.at[pl.ds(c * NP + ci * CHUNK, CHUNK)])

    return pl.kernel(body,
                     out_type=jax.ShapeDtypeStruct((NC * NP, H), jnp.float32),
                     mesh=mesh, scratch_types=scratch)


def _linear_body(agg_r, cnt_r, h_r, wl_r, bl_r, wr_r, z_r, sum_r, ssq_r):
    a = agg_r[0] + agg_r[1]
    cnt = cnt_r[0, :, 0:1] + cnt_r[1, :, 0:1]
    a = a / jnp.maximum(cnt, 1.0)
    z = (jnp.dot(a, wl_r[:, :], preferred_element_type=jnp.float32)
         + bl_r[:, :]
         + jnp.dot(h_r[:, :], wr_r[:, :], preferred_element_type=jnp.float32))
    z_r[:, :] = z

    @pl.when(pl.program_id(0) == 0)
    def _():
        sum_r[:, :] = jnp.zeros((8, H), jnp.float32)
        ssq_r[:, :] = jnp.zeros((8, H), jnp.float32)

    sum_r[0:1, :] += jnp.sum(z, axis=0, keepdims=True)
    ssq_r[0:1, :] += jnp.sum(z * z, axis=0, keepdims=True)


def _layer_linear(agg3, cnt3, h, wl, bl2, wr):
    nb = N // RB
    return pl.pallas_call(
        _linear_body,
        grid=(nb,),
        in_specs=[
            pl.BlockSpec((NC, RB, H), lambda i: (0, i, 0)),
            pl.BlockSpec((NC, RB, H), lambda i: (0, i, 0)),
            pl.BlockSpec((RB, H), lambda i: (i, 0)),
            pl.BlockSpec((H, H), lambda i: (0, 0)),
            pl.BlockSpec((1, H), lambda i: (0, 0)),
            pl.BlockSpec((H, H), lambda i: (0, 0)),
        ],
        out_specs=[
            pl.BlockSpec((RB, H), lambda i: (i, 0)),
            pl.BlockSpec((8, H), lambda i: (0, 0)),
            pl.BlockSpec((8, H), lambda i: (0, 0)),
        ],
        out_shape=[
            jax.ShapeDtypeStruct((N, H), jnp.float32),
            jax.ShapeDtypeStruct((8, H), jnp.float32),
            jax.ShapeDtypeStruct((8, H), jnp.float32),
        ],
    )(agg3, cnt3, h, wl, bl2, wr)


def _bn_stats(sum_r, ssq_r):
    mu = sum_r[0:1, :] * (1.0 / N)
    var = ssq_r[0:1, :] * (1.0 / N) - mu * mu
    return mu, lax.rsqrt(var + EPS)


def _bn_body(z_r, sum_r, ssq_r, g_r, b_r, o_r):
    mu, inv = _bn_stats(sum_r, ssq_r)
    o_r[:, :] = jnp.maximum(
        g_r[:, :] * (z_r[:, :] - mu) * inv + b_r[:, :], 0.0)


def _bn_relu(z, ssum, ssq, g2, b2):
    return pl.pallas_call(
        _bn_body,
        grid=(N // RB,),
        in_specs=[
            pl.BlockSpec((RB, H), lambda i: (i, 0)),
            pl.BlockSpec((8, H), lambda i: (0, 0)),
            pl.BlockSpec((8, H), lambda i: (0, 0)),
            pl.BlockSpec((1, H), lambda i: (0, 0)),
            pl.BlockSpec((1, H), lambda i: (0, 0)),
        ],
        out_specs=pl.BlockSpec((RB, H), lambda i: (i, 0)),
        out_shape=jax.ShapeDtypeStruct((N, H), jnp.float32),
    )(z, ssum, ssq, g2, b2)


def _bn_pool_body(z_r, sum_r, ssq_r, g_r, b_r, bt_r, ps_r, gc_r):
    mu, inv = _bn_stats(sum_r, ssq_r)
    hnew = jnp.maximum(g_r[:, :] * (z_r[:, :] - mu) * inv + b_r[:, :], 0.0)
    bt = bt_r[0]
    oh = (bt == lax.broadcasted_iota(jnp.int32, (G, 1), 0)).astype(jnp.float32)

    @pl.when(pl.program_id(0) == 0)
    def _():
        ps_r[:, :] = jnp.zeros((G, H), jnp.float32)
        gc_r[:, :] = jnp.zeros((G, H), jnp.float32)

    ps_r[:, :] += jnp.dot(oh, hnew, preferred_element_type=jnp.float32)
    gc_r[:, :] += jnp.broadcast_to(jnp.sum(oh, axis=1, keepdims=True), (G, H))


def _bn_pool(z, ssum, ssq, g2, b2, batch3):
    return pl.pallas_call(
        _bn_pool_body,
        grid=(N // RB,),
        in_specs=[
            pl.BlockSpec((RB, H), lambda i: (i, 0)),
            pl.BlockSpec((8, H), lambda i: (0, 0)),
            pl.BlockSpec((8, H), lambda i: (0, 0)),
            pl.BlockSpec((1, H), lambda i: (0, 0)),
            pl.BlockSpec((1, H), lambda i: (0, 0)),
            pl.BlockSpec((1, 1, RB), lambda i: (i, 0, 0)),
        ],
        out_specs=[
            pl.BlockSpec((G, H), lambda i: (0, 0)),
            pl.BlockSpec((G, H), lambda i: (0, 0)),
        ],
        out_shape=[
            jax.ShapeDtypeStruct((G, H), jnp.float32),
            jax.ShapeDtypeStruct((G, H), jnp.float32),
        ],
    )(z, ssum, ssq, g2, b2, batch3)


def _head_body(ps_r, gc_r, w1_r, b1_r, w2_r, b2_r, o_r):
    gp = ps_r[:, :] / jnp.maximum(gc_r[:, :], 1.0)
    z1 = jnp.maximum(
        jnp.dot(gp, w1_r[:, :], preferred_element_type=jnp.float32)
        + b1_r[:, :], 0.0)
    o_r[:, :] = (jnp.dot(z1, w2_r[:, :], preferred_element_type=jnp.float32)
                 + b2_r[:, :])


def _head(ps, gc, w1, b1, w2p, b2p):
    return pl.pallas_call(
        _head_body,
        out_shape=jax.ShapeDtypeStruct((G, H), jnp.float32),
    )(ps, gc, w1, b1, w2p, b2p)


def kernel(x, edge_index, batch, Wl, bl, Wr, gamma, beta, Wc1, bc1, Wc2, bc2):
    E = edge_index.shape[1]
    cpt = -(-E // (NC * NS * CHUNK))
    cpt = -(-cpt // 16) * 16
    ep = NC * NS * cpt * CHUNK
    src = jnp.concatenate(
        [edge_index[0], jnp.zeros((ep - E,), jnp.int32)]).reshape(-1, CHUNK)
    dst = jnp.concatenate(
        [edge_index[1], jnp.full((ep - E,), N, jnp.int32)]).reshape(-1, CHUNK)

    sc_agg = _make_sc_agg(cpt)

    bl2 = bl.reshape(L, 1, H)
    g2 = gamma.reshape(L, 1, H)
    b2 = beta.reshape(L, 1, H)
    batch3 = batch.reshape(N // RB, 1, RB)

    ones_rows = jnp.ones((8, H), jnp.float32)
    cnt3 = sc_agg(ones_rows, jnp.zeros_like(src), dst).reshape(NC, NP, H)
    h = x
    for i in range(L):
        agg = sc_agg(h, src, dst)
        agg3 = agg.reshape(NC, NP, H)
        z, ssum, ssq = _layer_linear(agg3, cnt3, h, Wl[i], bl2[i], Wr[i])
        if i < L - 1:
            h = _bn_relu(z, ssum, ssq, g2[i], b2[i])
        else:
            ps, gc = _bn_pool(z, ssum, ssq, g2[i], b2[i], batch3)

    w2p = jnp.pad(Wc2, ((0, 0), (0, H - C)))
    b2p = jnp.pad(bc2, (0, H - C)).reshape(1, H)
    out = _head(ps, gc, Wc1, bc1.reshape(1, H // 2), w2p, b2p)
    return out[:, :C]


# speedup vs baseline: 6.8352x; 6.8352x over previous
import functools

import jax
import jax.numpy as jnp
from jax import lax
from jax.experimental import pallas as pl
from jax.experimental.pallas import tpu as pltpu
from jax.experimental.pallas import tpu_sc as plsc

N = 10000
H = 128
L = 4
G = 64
C = 10

NC = 2
NS = 16
CHUNK = 128
NP = 10112
NCH = NP // CHUNK
ZPT = 5
RB = 1000
EPS = 1e-5


@functools.lru_cache(maxsize=None)
def _make_sc_cnt(cpt: int):
    mesh = plsc.VectorSubcoreMesh(core_axis_name="c", subcore_axis_name="s")
    scratch = [
        pltpu.VMEM((cpt, CHUNK), jnp.int32),
        pltpu.VMEM((CHUNK, H), jnp.float32),
        pltpu.VMEM_SHARED((NP, H), jnp.float32),
    ]

    def body(dst_hbm, cnt_hbm, dst_v, ones_v, cacc):
        c = lax.axis_index("c")
        s = lax.axis_index("s")
        wid = c * NS + s

        def fill(val):
            vv = jnp.full((16,), val, jnp.float32)

            def frow(i, carry):
                for q in range(H // 16):
                    ones_v[i, pl.ds(q * 16, 16)] = vv
                return carry

            lax.fori_loop(0, CHUNK, frow, 0)

        fill(0.0)
        for k in range(ZPT):
            ci = s * ZPT + k

            @pl.when(ci < NCH)
            def _():
                pltpu.sync_copy(ones_v, cacc.at[pl.ds(ci * CHUNK, CHUNK)])

        fill(1.0)
        pltpu.sync_copy(dst_hbm.at[pl.ds(wid * cpt, cpt)], dst_v)
        plsc.subcore_barrier()

        def step(i, carry):
            pltpu.sync_copy(ones_v, cacc.at[dst_v.at[i]], add=True)
            return carry

        lax.fori_loop(0, cpt, step, 0)
        plsc.subcore_barrier()
        for k in range(ZPT):
            ci = s * ZPT + k

            @pl.when(ci < NCH)
            def _():
                pltpu.sync_copy(
                    cacc.at[pl.ds(ci * CHUNK, CHUNK)],
                    cnt_hbm.at[pl.ds(c * NP + ci * CHUNK, CHUNK)])

    return pl.kernel(body,
                     out_type=jax.ShapeDtypeStruct((NC * NP, H), jnp.float32),
                     mesh=mesh, scratch_types=scratch)


@functools.lru_cache(maxsize=None)
def _make_sc_agg(cpt: int):
    mesh = plsc.VectorSubcoreMesh(core_axis_name="c", subcore_axis_name="s")
    hc = cpt // 2
    scratch = [
        pltpu.VMEM((hc, CHUNK), jnp.int32),
        pltpu.VMEM((hc, CHUNK), jnp.int32),
        pltpu.VMEM((CHUNK, H), jnp.float32),
        pltpu.VMEM((CHUNK, H), jnp.float32),
        pltpu.VMEM_SHARED((NP, H), jnp.float32),
        pltpu.SemaphoreType.DMA,
        pltpu.SemaphoreType.DMA,
    ]

    def body(h_hbm, src_hbm, dst_hbm, agg_hbm, src_v, dst_v, rows0, rows1,
             acc, sem0, sem1):
        c = lax.axis_index("c")
        s = lax.axis_index("s")
        wid = c * NS + s

        zv = jnp.zeros((16,), jnp.float32)

        def zero_row(i, carry):
            for q in range(H // 16):
                rows0[i, pl.ds(q * 16, 16)] = zv
            return carry

        lax.fori_loop(0, CHUNK, zero_row, 0)
        for k in range(ZPT):
            ci = s * ZPT + k

            @pl.when(ci < NCH)
            def _():
                pltpu.sync_copy(rows0, acc.at[pl.ds(ci * CHUNK, CHUNK)])

        plsc.subcore_barrier()

        def step(i, carry):
            a = 2 * i
            b = a + 1
            pltpu.async_copy(h_hbm.at[src_v.at[b]], rows1, sem1)
            pltpu.make_async_copy(h_hbm.at[src_v.at[a]], rows0, sem0).wait()
            pltpu.sync_copy(rows0, acc.at[dst_v.at[a]], add=True)

            @pl.when(b + 1 < hc)
            def _():
                pltpu.async_copy(h_hbm.at[src_v.at[b + 1]], rows0, sem0)

            pltpu.make_async_copy(h_hbm.at[src_v.at[b]], rows1, sem1).wait()
            pltpu.sync_copy(rows1, acc.at[dst_v.at[b]], add=True)
            return carry

        for p in range(2):
            pltpu.sync_copy(src_hbm.at[pl.ds(wid * cpt + p * hc, hc)], src_v)
            pltpu.sync_copy(dst_hbm.at[pl.ds(wid * cpt + p * hc, hc)], dst_v)
            pltpu.async_copy(h_hbm.at[src_v.at[0]], rows0, sem0)
            lax.fori_loop(0, hc // 2, step, 0)

        plsc.subcore_barrier()

        for k in range(ZPT):
            ci = s * ZPT + k

            @pl.when(ci < NCH)
            def _():
                pltpu.sync_copy(
                    acc.at[pl.ds(ci * CHUNK, CHUNK)],
                    agg_hbm.at[pl.ds(c * NP + ci * CHUNK, CHUNK)])

    return pl.kernel(body,
                     out_type=jax.ShapeDtypeStruct((NC * NP, H), jnp.float32),
                     mesh=mesh, scratch_types=scratch)


def _linear_body(agg_r, cnt_r, h_r, wl_r, bl_r, wr_r, z_r, sum_r, ssq_r):
    a = agg_r[0] + agg_r[1]
    cnt = cnt_r[0, :, 0:1] + cnt_r[1, :, 0:1]
    a = a / jnp.maximum(cnt, 1.0)
    z = (jnp.dot(a, wl_r[:, :], preferred_element_type=jnp.float32,
                 precision=lax.Precision.HIGHEST)
         + bl_r[:, :]
         + jnp.dot(h_r[:, :], wr_r[:, :], preferred_element_type=jnp.float32,
                   precision=lax.Precision.HIGHEST))
    z_r[:, :] = z

    @pl.when(pl.program_id(0) == 0)
    def _():
        sum_r[:, :] = jnp.zeros((8, H), jnp.float32)
        ssq_r[:, :] = jnp.zeros((8, H), jnp.float32)

    sum_r[0:1, :] += jnp.sum(z, axis=0, keepdims=True)
    ssq_r[0:1, :] += jnp.sum(z * z, axis=0, keepdims=True)


def _layer_linear(agg3, cnt3, h, wl, bl2, wr):
    nb = N // RB
    return pl.pallas_call(
        _linear_body,
        grid=(nb,),
        in_specs=[
            pl.BlockSpec((NC, RB, H), lambda i: (0, i, 0)),
            pl.BlockSpec((NC, RB, H), lambda i: (0, i, 0)),
            pl.BlockSpec((RB, H), lambda i: (i, 0)),
            pl.BlockSpec((H, H), lambda i: (0, 0)),
            pl.BlockSpec((1, H), lambda i: (0, 0)),
            pl.BlockSpec((H, H), lambda i: (0, 0)),
        ],
        out_specs=[
            pl.BlockSpec((RB, H), lambda i: (i, 0)),
            pl.BlockSpec((8, H), lambda i: (0, 0)),
            pl.BlockSpec((8, H), lambda i: (0, 0)),
        ],
        out_shape=[
            jax.ShapeDtypeStruct((N, H), jnp.float32),
            jax.ShapeDtypeStruct((8, H), jnp.float32),
            jax.ShapeDtypeStruct((8, H), jnp.float32),
        ],
    )(agg3, cnt3, h, wl, bl2, wr)


def _bn_stats(sum_r, ssq_r):
    mu = sum_r[0:1, :] * (1.0 / N)
    var = ssq_r[0:1, :] * (1.0 / N) - mu * mu
    return mu, lax.rsqrt(var + EPS)


def _bn_body(z_r, sum_r, ssq_r, g_r, b_r, o_r):
    mu, inv = _bn_stats(sum_r, ssq_r)
    o_r[:, :] = jnp.maximum(
        g_r[:, :] * (z_r[:, :] - mu) * inv + b_r[:, :], 0.0)


def _bn_relu(z, ssum, ssq, g2, b2):
    return pl.pallas_call(
        _bn_body,
        grid=(N // RB,),
        in_specs=[
            pl.BlockSpec((RB, H), lambda i: (i, 0)),
            pl.BlockSpec((8, H), lambda i: (0, 0)),
            pl.BlockSpec((8, H), lambda i: (0, 0)),
            pl.BlockSpec((1, H), lambda i: (0, 0)),
            pl.BlockSpec((1, H), lambda i: (0, 0)),
        ],
        out_specs=pl.BlockSpec((RB, H), lambda i: (i, 0)),
        out_shape=jax.ShapeDtypeStruct((N, H), jnp.float32),
    )(z, ssum, ssq, g2, b2)


def _bn_pool_body(z_r, sum_r, ssq_r, g_r, b_r, bt_r, ps_r, gc_r):
    mu, inv = _bn_stats(sum_r, ssq_r)
    hnew = jnp.maximum(g_r[:, :] * (z_r[:, :] - mu) * inv + b_r[:, :], 0.0)
    bt = bt_r[0]
    oh = (bt == lax.broadcasted_iota(jnp.int32, (G, 1), 0)).astype(jnp.float32)

    @pl.when(pl.program_id(0) == 0)
    def _():
        ps_r[:, :] = jnp.zeros((G, H), jnp.float32)
        gc_r[:, :] = jnp.zeros((G, H), jnp.float32)

    ps_r[:, :] += jnp.dot(oh, hnew, preferred_element_type=jnp.float32,
                          precision=lax.Precision.HIGHEST)
    gc_r[:, :] += jnp.broadcast_to(jnp.sum(oh, axis=1, keepdims=True), (G, H))


def _bn_pool(z, ssum, ssq, g2, b2, batch3):
    return pl.pallas_call(
        _bn_pool_body,
        grid=(N // RB,),
        in_specs=[
            pl.BlockSpec((RB, H), lambda i: (i, 0)),
            pl.BlockSpec((8, H), lambda i: (0, 0)),
            pl.BlockSpec((8, H), lambda i: (0, 0)),
            pl.BlockSpec((1, H), lambda i: (0, 0)),
            pl.BlockSpec((1, H), lambda i: (0, 0)),
            pl.BlockSpec((1, 1, RB), lambda i: (i, 0, 0)),
        ],
        out_specs=[
            pl.BlockSpec((G, H), lambda i: (0, 0)),
            pl.BlockSpec((G, H), lambda i: (0, 0)),
        ],
        out_shape=[
            jax.ShapeDtypeStruct((G, H), jnp.float32),
            jax.ShapeDtypeStruct((G, H), jnp.float32),
        ],
    )(z, ssum, ssq, g2, b2, batch3)


def _head_body(ps_r, gc_r, w1_r, b1_r, w2_r, b2_r, o_r):
    gp = ps_r[:, :] / jnp.maximum(gc_r[:, :], 1.0)
    z1 = jnp.maximum(
        jnp.dot(gp, w1_r[:, :], preferred_element_type=jnp.float32,
                precision=lax.Precision.HIGHEST)
        + b1_r[:, :], 0.0)
    o_r[:, :] = (jnp.dot(z1, w2_r[:, :], preferred_element_type=jnp.float32,
                         precision=lax.Precision.HIGHEST)
                 + b2_r[:, :])


def _head(ps, gc, w1, b1, w2p, b2p):
    return pl.pallas_call(
        _head_body,
        out_shape=jax.ShapeDtypeStruct((G, H), jnp.float32),
    )(ps, gc, w1, b1, w2p, b2p)


def kernel(x, edge_index, batch, Wl, bl, Wr, gamma, beta, Wc1, bc1, Wc2, bc2):
    E = edge_index.shape[1]
    cpt = -(-E // (NC * NS * CHUNK))
    cpt = -(-cpt // 16) * 16
    ep = NC * NS * cpt * CHUNK
    src = jnp.concatenate(
        [edge_index[0], jnp.zeros((ep - E,), jnp.int32)]).reshape(-1, CHUNK)
    dst = jnp.concatenate(
        [edge_index[1], jnp.full((ep - E,), N, jnp.int32)]).reshape(-1, CHUNK)

    sc_agg = _make_sc_agg(cpt)

    bl2 = bl.reshape(L, 1, H)
    g2 = gamma.reshape(L, 1, H)
    b2 = beta.reshape(L, 1, H)
    batch3 = batch.reshape(N // RB, 1, RB)

    cnt3 = _make_sc_cnt(cpt)(dst).reshape(NC, NP, H)
    h = x
    for i in range(L):
        agg = sc_agg(h, src, dst)
        agg3 = agg.reshape(NC, NP, H)
        z, ssum, ssq = _layer_linear(agg3, cnt3, h, Wl[i], bl2[i], Wr[i])
        if i < L - 1:
            h = _bn_relu(z, ssum, ssq, g2[i], b2[i])
        else:
            ps, gc = _bn_pool(z, ssum, ssq, g2[i], b2[i], batch3)

    w2p = jnp.pad(Wc2, ((0, 0), (0, H - C)))
    b2p = jnp.pad(bc2, (0, H - C)).reshape(1, H)
    out = _head(ps, gc, Wc1, bc1.reshape(1, H // 2), w2p, b2p)
    return out[:, :C]


# speedup vs baseline: 7.3537x; 1.0758x over previous
import functools

import jax
import jax.numpy as jnp
from jax import lax
from jax.experimental import pallas as pl
from jax.experimental.pallas import tpu as pltpu
from jax.experimental.pallas import tpu_sc as plsc

N = 10000
H = 128
L = 4
G = 64
C = 10

NC = 2
NS = 16
CHUNK = 128
NP = 10112
NCH = NP // CHUNK
ZPT = 5
RB = 1000
EPS = 1e-5


@functools.lru_cache(maxsize=None)
def _make_sc_cnt(cpt: int):
    mesh = plsc.VectorSubcoreMesh(core_axis_name="c", subcore_axis_name="s")
    scratch = [
        pltpu.VMEM((cpt, CHUNK), jnp.int32),
        pltpu.VMEM((CHUNK, H), jnp.float32),
        pltpu.VMEM_SHARED((NP, H), jnp.float32),
    ]

    def body(dst_hbm, cnt_hbm, dst_v, ones_v, cacc):
        c = lax.axis_index("c")
        s = lax.axis_index("s")
        wid = c * NS + s

        def fill(val):
            vv = jnp.full((16,), val, jnp.float32)

            def frow(i, carry):
                for q in range(H // 16):
                    ones_v[i, pl.ds(q * 16, 16)] = vv
                return carry

            lax.fori_loop(0, CHUNK, frow, 0)

        fill(0.0)
        for k in range(ZPT):
            ci = s * ZPT + k

            @pl.when(ci < NCH)
            def _():
                pltpu.sync_copy(ones_v, cacc.at[pl.ds(ci * CHUNK, CHUNK)])

        fill(1.0)
        pltpu.sync_copy(dst_hbm.at[pl.ds(wid * cpt, cpt)], dst_v)
        plsc.subcore_barrier()

        def step(i, carry):
            pltpu.sync_copy(ones_v, cacc.at[dst_v.at[i]], add=True)
            return carry

        lax.fori_loop(0, cpt, step, 0)
        plsc.subcore_barrier()
        for k in range(ZPT):
            ci = s * ZPT + k

            @pl.when(ci < NCH)
            def _():
                pltpu.sync_copy(
                    cacc.at[pl.ds(ci * CHUNK, CHUNK)],
                    cnt_hbm.at[pl.ds(c * NP + ci * CHUNK, CHUNK)])

    return pl.kernel(body,
                     out_type=jax.ShapeDtypeStruct((NC * NP, H), jnp.float32),
                     mesh=mesh, scratch_types=scratch)


@functools.lru_cache(maxsize=None)
def _make_sc_agg(cpt0: int, cpt1: int):
    mesh = plsc.VectorSubcoreMesh(core_axis_name="c", subcore_axis_name="s")
    hc0 = cpt0 // 2
    hc1 = cpt1 // 2
    scratch = [
        pltpu.VMEM((hc0, CHUNK), jnp.int32),
        pltpu.VMEM((hc0, CHUNK), jnp.int32),
        pltpu.VMEM((CHUNK, H), jnp.float32),
        pltpu.VMEM((CHUNK, H), jnp.float32),
        pltpu.VMEM_SHARED((NP, H), jnp.float32),
        pltpu.SemaphoreType.DMA,
        pltpu.SemaphoreType.DMA,
    ]

    def body(h_hbm, src_hbm, dst_hbm, agg_hbm, src_v, dst_v, rows0, rows1,
             acc, sem0, sem1):
        c = lax.axis_index("c")
        s = lax.axis_index("s")

        zv = jnp.zeros((16,), jnp.float32)

        def zero_row(i, carry):
            for q in range(H // 16):
                rows0[i, pl.ds(q * 16, 16)] = zv
            return carry

        lax.fori_loop(0, CHUNK, zero_row, 0)
        for k in range(ZPT):
            ci = s * ZPT + k

            @pl.when(ci < NCH)
            def _():
                pltpu.sync_copy(rows0, acc.at[pl.ds(ci * CHUNK, CHUNK)])

        plsc.subcore_barrier()

        def edge_phase(base, hc):
            def step(i, carry):
                a = 2 * i
                b = a + 1
                pltpu.async_copy(h_hbm.at[src_v.at[b]], rows1, sem1)
                pltpu.make_async_copy(h_hbm.at[src_v.at[a]], rows0,
                                      sem0).wait()
                pltpu.sync_copy(rows0, acc.at[dst_v.at[a]], add=True)

                @pl.when(b + 1 < hc)
                def _():
                    pltpu.async_copy(h_hbm.at[src_v.at[b + 1]], rows0, sem0)

                pltpu.make_async_copy(h_hbm.at[src_v.at[b]], rows1,
                                      sem1).wait()
                pltpu.sync_copy(rows1, acc.at[dst_v.at[b]], add=True)
                return carry

            for p in range(2):
                pltpu.sync_copy(src_hbm.at[pl.ds(base + p * hc, hc)],
                                src_v.at[pl.ds(0, hc)])
                pltpu.sync_copy(dst_hbm.at[pl.ds(base + p * hc, hc)],
                                dst_v.at[pl.ds(0, hc)])
                pltpu.async_copy(h_hbm.at[src_v.at[0]], rows0, sem0)
                lax.fori_loop(0, hc // 2, step, 0)

        @pl.when(c == 0)
        def _():
            edge_phase(s * cpt0, hc0)

        @pl.when(c == 1)
        def _():
            edge_phase(NS * cpt0 + s * cpt1, hc1)

        plsc.subcore_barrier()

        for k in range(ZPT):
            ci = s * ZPT + k

            @pl.when(ci < NCH)
            def _():
                pltpu.sync_copy(
                    acc.at[pl.ds(ci * CHUNK, CHUNK)],
                    agg_hbm.at[pl.ds(c * NP + ci * CHUNK, CHUNK)])

    return pl.kernel(body,
                     out_type=jax.ShapeDtypeStruct((NC * NP, H), jnp.float32),
                     mesh=mesh, scratch_types=scratch)


def _linear_body(agg_r, cnt_r, h_r, wl_r, bl_r, wr_r, z_r, sum_r, ssq_r):
    a = agg_r[0] + agg_r[1]
    cnt = cnt_r[0, :, 0:1] + cnt_r[1, :, 0:1]
    a = a / jnp.maximum(cnt, 1.0)
    z = (jnp.dot(a, wl_r[:, :], preferred_element_type=jnp.float32,
                 precision=lax.Precision.HIGHEST)
         + bl_r[:, :]
         + jnp.dot(h_r[:, :], wr_r[:, :], preferred_element_type=jnp.float32,
                   precision=lax.Precision.HIGHEST))
    z_r[:, :] = z

    @pl.when(pl.program_id(0) == 0)
    def _():
        sum_r[:, :] = jnp.zeros((8, H), jnp.float32)
        ssq_r[:, :] = jnp.zeros((8, H), jnp.float32)

    sum_r[0:1, :] += jnp.sum(z, axis=0, keepdims=True)
    ssq_r[0:1, :] += jnp.sum(z * z, axis=0, keepdims=True)


def _layer_linear(agg3, cnt3, h, wl, bl2, wr):
    nb = N // RB
    return pl.pallas_call(
        _linear_body,
        grid=(nb,),
        in_specs=[
            pl.BlockSpec((NC, RB, H), lambda i: (0, i, 0)),
            pl.BlockSpec((NC, RB, H), lambda i: (0, i, 0)),
            pl.BlockSpec((RB, H), lambda i: (i, 0)),
            pl.BlockSpec((H, H), lambda i: (0, 0)),
            pl.BlockSpec((1, H), lambda i: (0, 0)),
            pl.BlockSpec((H, H), lambda i: (0, 0)),
        ],
        out_specs=[
            pl.BlockSpec((RB, H), lambda i: (i, 0)),
            pl.BlockSpec((8, H), lambda i: (0, 0)),
            pl.BlockSpec((8, H), lambda i: (0, 0)),
        ],
        out_shape=[
            jax.ShapeDtypeStruct((N, H), jnp.float32),
            jax.ShapeDtypeStruct((8, H), jnp.float32),
            jax.ShapeDtypeStruct((8, H), jnp.float32),
        ],
    )(agg3, cnt3, h, wl, bl2, wr)


def _bn_stats(sum_r, ssq_r):
    mu = sum_r[0:1, :] * (1.0 / N)
    var = ssq_r[0:1, :] * (1.0 / N) - mu * mu
    return mu, lax.rsqrt(var + EPS)


def _bn_body(z_r, sum_r, ssq_r, g_r, b_r, o_r):
    mu, inv = _bn_stats(sum_r, ssq_r)
    o_r[:, :] = jnp.maximum(
        g_r[:, :] * (z_r[:, :] - mu) * inv + b_r[:, :], 0.0)


def _bn_relu(z, ssum, ssq, g2, b2):
    return pl.pallas_call(
        _bn_body,
        grid=(N // RB,),
        in_specs=[
            pl.BlockSpec((RB, H), lambda i: (i, 0)),
            pl.BlockSpec((8, H), lambda i: (0, 0)),
            pl.BlockSpec((8, H), lambda i: (0, 0)),
            pl.BlockSpec((1, H), lambda i: (0, 0)),
            pl.BlockSpec((1, H), lambda i: (0, 0)),
        ],
        out_specs=pl.BlockSpec((RB, H), lambda i: (i, 0)),
        out_shape=jax.ShapeDtypeStruct((N, H), jnp.float32),
    )(z, ssum, ssq, g2, b2)


def _bn_pool_body(z_r, sum_r, ssq_r, g_r, b_r, bt_r, ps_r, gc_r):
    mu, inv = _bn_stats(sum_r, ssq_r)
    hnew = jnp.maximum(g_r[:, :] * (z_r[:, :] - mu) * inv + b_r[:, :], 0.0)
    bt = bt_r[0]
    oh = (bt == lax.broadcasted_iota(jnp.int32, (G, 1), 0)).astype(jnp.float32)

    @pl.when(pl.program_id(0) == 0)
    def _():
        ps_r[:, :] = jnp.zeros((G, H), jnp.float32)
        gc_r[:, :] = jnp.zeros((G, H), jnp.float32)

    ps_r[:, :] += jnp.dot(oh, hnew, preferred_element_type=jnp.float32,
                          precision=lax.Precision.HIGHEST)
    gc_r[:, :] += jnp.broadcast_to(jnp.sum(oh, axis=1, keepdims=True), (G, H))


def _bn_pool(z, ssum, ssq, g2, b2, batch3):
    return pl.pallas_call(
        _bn_pool_body,
        grid=(N // RB,),
        in_specs=[
            pl.BlockSpec((RB, H), lambda i: (i, 0)),
            pl.BlockSpec((8, H), lambda i: (0, 0)),
            pl.BlockSpec((8, H), lambda i: (0, 0)),
            pl.BlockSpec((1, H), lambda i: (0, 0)),
            pl.BlockSpec((1, H), lambda i: (0, 0)),
            pl.BlockSpec((1, 1, RB), lambda i: (i, 0, 0)),
        ],
        out_specs=[
            pl.BlockSpec((G, H), lambda i: (0, 0)),
            pl.BlockSpec((G, H), lambda i: (0, 0)),
        ],
        out_shape=[
            jax.ShapeDtypeStruct((G, H), jnp.float32),
            jax.ShapeDtypeStruct((G, H), jnp.float32),
        ],
    )(z, ssum, ssq, g2, b2, batch3)


def _head_body(ps_r, gc_r, w1_r, b1_r, w2_r, b2_r, o_r):
    gp = ps_r[:, :] / jnp.maximum(gc_r[:, :], 1.0)
    z1 = jnp.maximum(
        jnp.dot(gp, w1_r[:, :], preferred_element_type=jnp.float32,
                precision=lax.Precision.HIGHEST)
        + b1_r[:, :], 0.0)
    o_r[:, :] = (jnp.dot(z1, w2_r[:, :], preferred_element_type=jnp.float32,
                         precision=lax.Precision.HIGHEST)
                 + b2_r[:, :])


def _head(ps, gc, w1, b1, w2p, b2p):
    return pl.pallas_call(
        _head_body,
        out_shape=jax.ShapeDtypeStruct((G, H), jnp.float32),
    )(ps, gc, w1, b1, w2p, b2p)


def kernel(x, edge_index, batch, Wl, bl, Wr, gamma, beta, Wc1, bc1, Wc2, bc2):
    E = edge_index.shape[1]
    cpt = -(-E // (NC * NS * CHUNK))
    cpt = -(-cpt // 16) * 16
    ep = NC * NS * cpt * CHUNK
    src = jnp.concatenate(
        [edge_index[0], jnp.zeros((ep - E,), jnp.int32)]).reshape(-1, CHUNK)
    dst = jnp.concatenate(
        [edge_index[1], jnp.full((ep - E,), N, jnp.int32)]).reshape(-1, CHUNK)

    tpp = 2 * cpt
    cpt0 = min(max((tpp * 4 // 5) // 16 * 16, 16), tpp - 16)
    cpt1 = tpp - cpt0
    sc_agg = _make_sc_agg(cpt0, cpt1)

    bl2 = bl.reshape(L, 1, H)
    g2 = gamma.reshape(L, 1, H)
    b2 = beta.reshape(L, 1, H)
    batch3 = batch.reshape(N // RB, 1, RB)

    cnt3 = _make_sc_cnt(cpt)(dst).reshape(NC, NP, H)
    h = x
    for i in range(L):
        agg = sc_agg(h, src, dst)
        agg3 = agg.reshape(NC, NP, H)
        z, ssum, ssq = _layer_linear(agg3, cnt3, h, Wl[i], bl2[i], Wr[i])
        if i < L - 1:
            h = _bn_relu(z, ssum, ssq, g2[i], b2[i])
        else:
            ps, gc = _bn_pool(z, ssum, ssq, g2[i], b2[i], batch3)

    w2p = jnp.pad(Wc2, ((0, 0), (0, H - C)))
    b2p = jnp.pad(bc2, (0, H - C)).reshape(1, H)
    out = _head(ps, gc, Wc1, bc1.reshape(1, H // 2), w2p, b2p)
    return out[:, :C]


# speedup vs baseline: 8.3699x; 1.1382x over previous
import functools

import jax
import jax.numpy as jnp
from jax import lax
from jax.experimental import pallas as pl
from jax.experimental.pallas import tpu as pltpu
from jax.experimental.pallas import tpu_sc as plsc

N = 10000
H = 128
L = 4
G = 64
C = 10

NC = 2
NS = 16
CHUNK = 128
NP = 10112
NCH = NP // CHUNK
ZPT = 5
RB = 1000
EPS = 1e-5


def _slab(cptc: int) -> int:
    if cptc == 0:
        return 8
    for h in (48, 40, 32, 24, 16, 8):
        if cptc % h == 0:
            return h
    return 8


@functools.lru_cache(maxsize=None)
def _make_sc_cnt(cpt: int):
    mesh = plsc.VectorSubcoreMesh(core_axis_name="c", subcore_axis_name="s")
    scratch = [
        pltpu.VMEM((cpt, CHUNK), jnp.int32),
        pltpu.VMEM((CHUNK, H), jnp.float32),
        pltpu.VMEM_SHARED((NP, H), jnp.float32),
    ]

    def body(dst_hbm, cnt_hbm, dst_v, ones_v, cacc):
        c = lax.axis_index("c")
        s = lax.axis_index("s")
        wid = c * NS + s

        def fill(val):
            vv = jnp.full((16,), val, jnp.float32)

            def frow(i, carry):
                for q in range(H // 16):
                    ones_v[i, pl.ds(q * 16, 16)] = vv
                return carry

            lax.fori_loop(0, CHUNK, frow, 0)

        fill(0.0)
        for k in range(ZPT):
            ci = s * ZPT + k

            @pl.when(ci < NCH)
            def _():
                pltpu.sync_copy(ones_v, cacc.at[pl.ds(ci * CHUNK, CHUNK)])

        fill(1.0)
        pltpu.sync_copy(dst_hbm.at[pl.ds(wid * cpt, cpt)], dst_v)
        plsc.subcore_barrier()

        def step(i, carry):
            pltpu.sync_copy(ones_v, cacc.at[dst_v.at[i]], add=True)
            return carry

        lax.fori_loop(0, cpt, step, 0)
        plsc.subcore_barrier()
        for k in range(ZPT):
            ci = s * ZPT + k

            @pl.when(ci < NCH)
            def _():
                pltpu.sync_copy(
                    cacc.at[pl.ds(ci * CHUNK, CHUNK)],
                    cnt_hbm.at[pl.ds(c * NP + ci * CHUNK, CHUNK)])

    return pl.kernel(body,
                     out_type=jax.ShapeDtypeStruct((NC * NP, H), jnp.float32),
                     mesh=mesh, scratch_types=scratch)


@functools.lru_cache(maxsize=None)
def _make_sc_agg(cpt0: int, cpt1: int):
    mesh = plsc.VectorSubcoreMesh(core_axis_name="c", subcore_axis_name="s")
    hc0 = _slab(cpt0)
    hc1 = _slab(cpt1)
    hmax = max(hc0, hc1)
    scratch = [
        pltpu.VMEM((hmax, CHUNK), jnp.int32),
        pltpu.VMEM((hmax, CHUNK), jnp.int32),
        pltpu.VMEM((CHUNK, H), jnp.float32),
        pltpu.VMEM((CHUNK, H), jnp.float32),
        pltpu.VMEM_SHARED((NP, H), jnp.float32),
        pltpu.SemaphoreType.DMA,
        pltpu.SemaphoreType.DMA,
    ]

    def body(h_hbm, src_hbm, dst_hbm, agg_hbm, src_v, dst_v, rows0, rows1,
             acc, sem0, sem1):
        c = lax.axis_index("c")
        s = lax.axis_index("s")

        zv = jnp.zeros((16,), jnp.float32)

        def zero_row(i, carry):
            for q in range(H // 16):
                rows0[i, pl.ds(q * 16, 16)] = zv
            return carry

        lax.fori_loop(0, CHUNK, zero_row, 0)
        for k in range(ZPT):
            ci = s * ZPT + k

            @pl.when(ci < NCH)
            def _():
                pltpu.sync_copy(rows0, acc.at[pl.ds(ci * CHUNK, CHUNK)])

        plsc.subcore_barrier()

        def edge_phase(base, hc, nslab):
            def step(i, carry):
                a = 2 * i
                b = a + 1
                pltpu.async_copy(h_hbm.at[src_v.at[b]], rows1, sem1)
                pltpu.make_async_copy(h_hbm.at[src_v.at[a]], rows0,
                                      sem0).wait()
                pltpu.sync_copy(rows0, acc.at[dst_v.at[a]], add=True)

                @pl.when(b + 1 < hc)
                def _():
                    pltpu.async_copy(h_hbm.at[src_v.at[b + 1]], rows0, sem0)

                pltpu.make_async_copy(h_hbm.at[src_v.at[b]], rows1,
                                      sem1).wait()
                pltpu.sync_copy(rows1, acc.at[dst_v.at[b]], add=True)
                return carry

            for p in range(nslab):
                pltpu.sync_copy(src_hbm.at[pl.ds(base + p * hc, hc)],
                                src_v.at[pl.ds(0, hc)])
                pltpu.sync_copy(dst_hbm.at[pl.ds(base + p * hc, hc)],
                                dst_v.at[pl.ds(0, hc)])
                pltpu.async_copy(h_hbm.at[src_v.at[0]], rows0, sem0)
                lax.fori_loop(0, hc // 2, step, 0)

        if cpt0:
            @pl.when(c == 0)
            def _():
                edge_phase(s * cpt0, hc0, cpt0 // hc0)

        if cpt1:
            @pl.when(c == 1)
            def _():
                edge_phase(NS * cpt0 + s * cpt1, hc1, cpt1 // hc1)

        plsc.subcore_barrier()

        for k in range(ZPT):
            ci = s * ZPT + k

            @pl.when(ci < NCH)
            def _():
                pltpu.sync_copy(
                    acc.at[pl.ds(ci * CHUNK, CHUNK)],
                    agg_hbm.at[pl.ds(c * NP + ci * CHUNK, CHUNK)])

    return pl.kernel(body,
                     out_type=jax.ShapeDtypeStruct((NC * NP, H), jnp.float32),
                     mesh=mesh, scratch_types=scratch)


def _linear_body(agg_r, cnt_r, h_r, wl_r, bl_r, wr_r, z_r, sum_r, ssq_r):
    a = agg_r[0] + agg_r[1]
    cnt = cnt_r[0, :, 0:1] + cnt_r[1, :, 0:1]
    a = a / jnp.maximum(cnt, 1.0)
    z = (jnp.dot(a, wl_r[:, :], preferred_element_type=jnp.float32,
                 precision=lax.Precision.HIGHEST)
         + bl_r[:, :]
         + jnp.dot(h_r[:, :], wr_r[:, :], preferred_element_type=jnp.float32,
                   precision=lax.Precision.HIGHEST))
    z_r[:, :] = z

    @pl.when(pl.program_id(0) == 0)
    def _():
        sum_r[:, :] = jnp.zeros((8, H), jnp.float32)
        ssq_r[:, :] = jnp.zeros((8, H), jnp.float32)

    sum_r[0:1, :] += jnp.sum(z, axis=0, keepdims=True)
    ssq_r[0:1, :] += jnp.sum(z * z, axis=0, keepdims=True)


def _layer_linear(agg3, cnt3, h, wl, bl2, wr):
    nb = N // RB
    return pl.pallas_call(
        _linear_body,
        grid=(nb,),
        in_specs=[
            pl.BlockSpec((NC, RB, H), lambda i: (0, i, 0)),
            pl.BlockSpec((NC, RB, H), lambda i: (0, i, 0)),
            pl.BlockSpec((RB, H), lambda i: (i, 0)),
            pl.BlockSpec((H, H), lambda i: (0, 0)),
            pl.BlockSpec((1, H), lambda i: (0, 0)),
            pl.BlockSpec((H, H), lambda i: (0, 0)),
        ],
        out_specs=[
            pl.BlockSpec((RB, H), lambda i: (i, 0)),
            pl.BlockSpec((8, H), lambda i: (0, 0)),
            pl.BlockSpec((8, H), lambda i: (0, 0)),
        ],
        out_shape=[
            jax.ShapeDtypeStruct((N, H), jnp.float32),
            jax.ShapeDtypeStruct((8, H), jnp.float32),
            jax.ShapeDtypeStruct((8, H), jnp.float32),
        ],
    )(agg3, cnt3, h, wl, bl2, wr)


def _bn_stats(sum_r, ssq_r):
    mu = sum_r[0:1, :] * (1.0 / N)
    var = ssq_r[0:1, :] * (1.0 / N) - mu * mu
    return mu, lax.rsqrt(var + EPS)


def _bn_body(z_r, sum_r, ssq_r, g_r, b_r, o_r):
    mu, inv = _bn_stats(sum_r, ssq_r)
    o_r[:, :] = jnp.maximum(
        g_r[:, :] * (z_r[:, :] - mu) * inv + b_r[:, :], 0.0)


def _bn_relu(z, ssum, ssq, g2, b2):
    return pl.pallas_call(
        _bn_body,
        grid=(N // RB,),
        in_specs=[
            pl.BlockSpec((RB, H), lambda i: (i, 0)),
            pl.BlockSpec((8, H), lambda i: (0, 0)),
            pl.BlockSpec((8, H), lambda i: (0, 0)),
            pl.BlockSpec((1, H), lambda i: (0, 0)),
            pl.BlockSpec((1, H), lambda i: (0, 0)),
        ],
        out_specs=pl.BlockSpec((RB, H), lambda i: (i, 0)),
        out_shape=jax.ShapeDtypeStruct((N, H), jnp.float32),
    )(z, ssum, ssq, g2, b2)


def _bn_pool_body(z_r, sum_r, ssq_r, g_r, b_r, bt_r, ps_r, gc_r):
    mu, inv = _bn_stats(sum_r, ssq_r)
    hnew = jnp.maximum(g_r[:, :] * (z_r[:, :] - mu) * inv + b_r[:, :], 0.0)
    bt = bt_r[0]
    oh = (bt == lax.broadcasted_iota(jnp.int32, (G, 1), 0)).astype(jnp.float32)

    @pl.when(pl.program_id(0) == 0)
    def _():
        ps_r[:, :] = jnp.zeros((G, H), jnp.float32)
        gc_r[:, :] = jnp.zeros((G, H), jnp.float32)

    ps_r[:, :] += jnp.dot(oh, hnew, preferred_element_type=jnp.float32,
                          precision=lax.Precision.HIGHEST)
    gc_r[:, :] += jnp.broadcast_to(jnp.sum(oh, axis=1, keepdims=True), (G, H))


def _bn_pool(z, ssum, ssq, g2, b2, batch3):
    return pl.pallas_call(
        _bn_pool_body,
        grid=(N // RB,),
        in_specs=[
            pl.BlockSpec((RB, H), lambda i: (i, 0)),
            pl.BlockSpec((8, H), lambda i: (0, 0)),
            pl.BlockSpec((8, H), lambda i: (0, 0)),
            pl.BlockSpec((1, H), lambda i: (0, 0)),
            pl.BlockSpec((1, H), lambda i: (0, 0)),
            pl.BlockSpec((1, 1, RB), lambda i: (i, 0, 0)),
        ],
        out_specs=[
            pl.BlockSpec((G, H), lambda i: (0, 0)),
            pl.BlockSpec((G, H), lambda i: (0, 0)),
        ],
        out_shape=[
            jax.ShapeDtypeStruct((G, H), jnp.float32),
            jax.ShapeDtypeStruct((G, H), jnp.float32),
        ],
    )(z, ssum, ssq, g2, b2, batch3)


def _head_body(ps_r, gc_r, w1_r, b1_r, w2_r, b2_r, o_r):
    gp = ps_r[:, :] / jnp.maximum(gc_r[:, :], 1.0)
    z1 = jnp.maximum(
        jnp.dot(gp, w1_r[:, :], preferred_element_type=jnp.float32,
                precision=lax.Precision.HIGHEST)
        + b1_r[:, :], 0.0)
    o_r[:, :] = (jnp.dot(z1, w2_r[:, :], preferred_element_type=jnp.float32,
                         precision=lax.Precision.HIGHEST)
                 + b2_r[:, :])


def _head(ps, gc, w1, b1, w2p, b2p):
    return pl.pallas_call(
        _head_body,
        out_shape=jax.ShapeDtypeStruct((G, H), jnp.float32),
    )(ps, gc, w1, b1, w2p, b2p)


def kernel(x, edge_index, batch, Wl, bl, Wr, gamma, beta, Wc1, bc1, Wc2, bc2):
    E = edge_index.shape[1]
    cpt = -(-E // (NC * NS * CHUNK))
    cpt = -(-cpt // 16) * 16
    ep = NC * NS * cpt * CHUNK
    src = jnp.concatenate(
        [edge_index[0], jnp.zeros((ep - E,), jnp.int32)]).reshape(-1, CHUNK)
    dst = jnp.concatenate(
        [edge_index[1], jnp.full((ep - E,), N, jnp.int32)]).reshape(-1, CHUNK)

    tpp = 2 * cpt
    cpt0 = min(max((tpp * 9 // 10) // 16 * 16, 16), tpp - 16)
    cpt1 = tpp - cpt0
    sc_agg = _make_sc_agg(cpt0, cpt1)

    bl2 = bl.reshape(L, 1, H)
    g2 = gamma.reshape(L, 1, H)
    b2 = beta.reshape(L, 1, H)
    batch3 = batch.reshape(N // RB, 1, RB)

    cnt3 = _make_sc_cnt(cpt)(dst).reshape(NC, NP, H)
    h = x
    for i in range(L):
        agg = sc_agg(h, src, dst)
        agg3 = agg.reshape(NC, NP, H)
        z, ssum, ssq = _layer_linear(agg3, cnt3, h, Wl[i], bl2[i], Wr[i])
        if i < L - 1:
            h = _bn_relu(z, ssum, ssq, g2[i], b2[i])
        else:
            ps, gc = _bn_pool(z, ssum, ssq, g2[i], b2[i], batch3)

    w2p = jnp.pad(Wc2, ((0, 0), (0, H - C)))
    b2p = jnp.pad(bc2, (0, H - C)).reshape(1, H)
    out = _head(ps, gc, Wc1, bc1.reshape(1, H // 2), w2p, b2p)
    return out[:, :C]
